# baseline (speedup 1.0000x reference)
import functools

import jax
import jax.numpy as jnp
from jax import lax
from jax.experimental import pallas as pl
from jax.experimental.pallas import tpu as pltpu
from jax.experimental.pallas import tpu_sc as plsc

RADIUS = 0.1
KERNEL_SIZE = 3
DILATION = 0.05
NDIM = 3
NCELLS = KERNEL_SIZE ** NDIM


def _make_sc_gather(n_idx, cw, lw):
    info = plsc.get_sparse_core_info()
    nc, ns = info.num_cores, info.num_subcores
    nw = nc * ns
    ch = 128
    assert n_idx % (nw * ch) == 0
    per_w = n_idx // nw
    iters = per_w // ch
    mesh = plsc.VectorSubcoreMesh(core_axis_name="c", subcore_axis_name="s")

    @functools.partial(
        pl.kernel,
        mesh=mesh,
        out_type=(jax.ShapeDtypeStruct((n_idx, cw), jnp.float32),
                  jax.ShapeDtypeStruct((n_idx, lw), jnp.float32)),
        scratch_types=[
            pltpu.VMEM((ch,), jnp.int32),
            pltpu.VMEM((ch, cw), jnp.float32),
            pltpu.VMEM((ch, lw), jnp.float32),
            pltpu.SemaphoreType.DMA,
            pltpu.SemaphoreType.DMA,
        ],
        compiler_params=pltpu.CompilerParams(use_tc_tiling_on_sc=False),
    )
    def gather(data_hbm, ploc_hbm, idx_hbm, gd_hbm, gl_hbm,
               idx_v, rows_d, rows_l, sem_d, sem_l):
        wid = lax.axis_index("s") * nc + lax.axis_index("c")
        base = wid * per_w

        def body(i, carry):
            off = base + i * ch
            pltpu.sync_copy(idx_hbm.at[pl.ds(off, ch)], idx_v)
            cp_d = pltpu.async_copy(data_hbm.at[idx_v], rows_d, sem_d)
            cp_l = pltpu.async_copy(ploc_hbm.at[idx_v], rows_l, sem_l)
            cp_d.wait()
            cp_l.wait()
            pltpu.sync_copy(rows_d, gd_hbm.at[pl.ds(off, ch)])
            pltpu.sync_copy(rows_l, gl_hbm.at[pl.ds(off, ch)])
            return carry

        lax.fori_loop(0, iters, body, 0)

    return gather


def _tc_body(k, c, mb, gd_ref, gl_ref, q_ref, nb_ref, w2_ref, b_ref, o_ref):
    norm = 1.0 / (RADIUS ** 3)
    half = (KERNEL_SIZE - 1) / 2.0
    ci = lax.broadcasted_iota(jnp.int32, (1, 32), 1)
    ox = ((ci // (KERNEL_SIZE * KERNEL_SIZE)).astype(jnp.float32) - half) * DILATION
    oy = (((ci // KERNEL_SIZE) % KERNEL_SIZE).astype(jnp.float32) - half) * DILATION
    oz = ((ci % KERNEL_SIZE).astype(jnp.float32) - half) * DILATION

    dg = gd_ref[...]
    far = jnp.where(nb_ref[...] < 0, 1e3, 0.0).astype(jnp.float32)
    d0 = (q_ref[:, 0:1] + far) - gl_ref[:, 0:1]
    d1 = q_ref[:, 1:2] - gl_ref[:, 1:2]
    d2 = q_ref[:, 2:3] - gl_ref[:, 2:3]
    e0 = d0 + ox
    e1 = d1 + oy
    e2 = d2 + oz
    dist2 = e0 * e0 + e1 * e1 + e2 * e2
    dist = jnp.sqrt(dist2 + 1e-12)
    t = jnp.maximum(1.0 - dist * (1.0 / RADIUS), 0.0)
    w = norm * t * t * t

    nc32 = w.shape[1]
    wide = NCELLS * c
    widep = 128 * ((wide + 127) // 128)
    row_i = lax.broadcasted_iota(jnp.int32, (nc32, widep), 0)
    col_i = lax.broadcasted_iota(jnp.int32, (nc32, widep), 1)
    rep = jnp.where((col_i // c == row_i) & (col_i < wide), 1.0, 0.0)
    rep = rep.astype(jnp.float32)

    hi = w.astype(jnp.bfloat16).astype(jnp.float32)
    lo = w - hi
    wexp = (jnp.dot(hi, rep, preferred_element_type=jnp.float32)
            + jnp.dot(lo, rep, preferred_element_type=jnp.float32))

    dg4 = jnp.concatenate([dg, dg, dg, dg], axis=1)
    dgt = jnp.concatenate([dg4] * (widep // 128), axis=1)

    p = wexp * dgt
    interp = p.reshape(mb, k, widep).sum(axis=1)
    acc = jnp.dot(interp, w2_ref[...], preferred_element_type=jnp.float32,
                  precision=jax.lax.Precision.HIGHEST)
    o_ref[...] = acc + b_ref[...]


def kernel(locs, data, neighbors, qlocs, weight, bias):
    b, n, d = locs.shape
    _, m, k = neighbors.shape
    c = data.shape[2]
    o = weight.shape[0]
    lw = 16
    r = b * m * k
    bm = b * m

    data2 = data.reshape(b * n, c)
    ploc = jnp.concatenate(
        [locs, jnp.zeros((b, n, lw - d), jnp.float32)], axis=-1).reshape(b * n, lw)

    nb = neighbors.astype(jnp.int32)
    base = (jnp.arange(b, dtype=jnp.int32) * n)[:, None, None]
    flat_idx = (jnp.clip(nb, 0, n - 1) + base).reshape(r)
    nbf = nb.reshape(r, 1)

    q4 = jnp.concatenate([qlocs, jnp.zeros((b, m, 1), jnp.float32)], axis=-1)
    qrep = jnp.broadcast_to(q4[:, :, None, :], (b, m, k, 4)).reshape(r, 4)

    gd, gl = _make_sc_gather(r, c, lw)(data2, ploc, flat_idx)

    w2p = jnp.transpose(weight, (2, 1, 0)).reshape(NCELLS * c, o)
    widep = 128 * ((NCELLS * c + 127) // 128)
    w2p = jnp.concatenate(
        [w2p, jnp.zeros((widep - NCELLS * c, o), jnp.float32)], axis=0)
    mb = 128
    out2 = pl.pallas_call(
        functools.partial(_tc_body, k, c, mb),
        grid=(bm // mb,),
        in_specs=[
            pl.BlockSpec((mb * k, c), lambda i: (i, 0)),
            pl.BlockSpec((mb * k, lw), lambda i: (i, 0)),
            pl.BlockSpec((mb * k, 4), lambda i: (i, 0)),
            pl.BlockSpec((mb * k, 1), lambda i: (i, 0)),
            pl.BlockSpec((widep, o), lambda i: (0, 0)),
            pl.BlockSpec((1, o), lambda i: (0, 0)),
        ],
        out_specs=pl.BlockSpec((mb, o), lambda i: (i, 0)),
        out_shape=jax.ShapeDtypeStruct((bm, o), jnp.float32),
    )(gd, gl, qrep, nbf, w2p, bias.reshape(1, o))

    return out2.reshape(b, m, o)


# speedup vs baseline: 1.0097x; 1.0097x over previous
import functools

import jax
import jax.numpy as jnp
from jax import lax
from jax.experimental import pallas as pl
from jax.experimental.pallas import tpu as pltpu
from jax.experimental.pallas import tpu_sc as plsc

RADIUS = 0.1
KERNEL_SIZE = 3
DILATION = 0.05
NDIM = 3
NCELLS = KERNEL_SIZE ** NDIM


def _make_sc_gather(n_idx, cw, lw):
    info = plsc.get_sparse_core_info()
    nc, ns = info.num_cores, info.num_subcores
    nw = nc * ns
    ch = 128
    assert n_idx % (nw * ch) == 0
    per_w = n_idx // nw
    iters = per_w // ch
    mesh = plsc.VectorSubcoreMesh(core_axis_name="c", subcore_axis_name="s")

    nbuf = 8
    assert iters % nbuf == 0

    @functools.partial(
        pl.kernel,
        mesh=mesh,
        out_type=(jax.ShapeDtypeStruct((n_idx, cw), jnp.float32),
                  jax.ShapeDtypeStruct((n_idx, lw), jnp.float32)),
        scratch_types=[
            pltpu.VMEM((per_w,), jnp.int32),
            pltpu.VMEM((nbuf, ch, cw), jnp.float32),
            pltpu.VMEM((nbuf, ch, lw), jnp.float32),
        ] + [pltpu.SemaphoreType.DMA] * (2 * nbuf),
        compiler_params=pltpu.CompilerParams(use_tc_tiling_on_sc=False),
    )
    def gather(data_hbm, ploc_hbm, idx_hbm, gd_hbm, gl_hbm,
               idx_v, rows_d, rows_l, *sems):
        gsem = sems[0:nbuf]
        ssem = sems[nbuf:2 * nbuf]
        wid = lax.axis_index("s") * nc + lax.axis_index("c")
        base = wid * per_w
        pltpu.sync_copy(idx_hbm.at[pl.ds(base, per_w)], idx_v)

        def issue(i, s):
            sl = idx_v.at[pl.ds(i * ch, ch)]
            pltpu.async_copy(data_hbm.at[sl], rows_d.at[s], gsem[s])
            pltpu.async_copy(ploc_hbm.at[sl], rows_l.at[s], gsem[s])

        for s in range(nbuf):
            issue(s, s)

        def blk(ib, carry):
            i0 = ib * nbuf
            for s in range(nbuf):
                i = i0 + s
                sl = idx_v.at[pl.ds(i * ch, ch)]
                pltpu.make_async_copy(data_hbm.at[sl], rows_d.at[s], gsem[s]).wait()
                pltpu.make_async_copy(ploc_hbm.at[sl], rows_l.at[s], gsem[s]).wait()
                st_d = pltpu.async_copy(
                    rows_d.at[s], gd_hbm.at[pl.ds(base + i * ch, ch)], ssem[s])
                st_l = pltpu.async_copy(
                    rows_l.at[s], gl_hbm.at[pl.ds(base + i * ch, ch)], ssem[s])
                st_d.wait()
                st_l.wait()

                @pl.when(ib < (iters // nbuf) - 1)
                def _():
                    issue(i + nbuf, s)
            return carry

        lax.fori_loop(0, iters // nbuf, blk, 0)

    return gather


def _tc_body(k, c, mb, gd_ref, gl_ref, q_ref, nb_ref, w2_ref, b_ref, o_ref):
    norm = 1.0 / (RADIUS ** 3)
    half = (KERNEL_SIZE - 1) / 2.0
    ci = lax.broadcasted_iota(jnp.int32, (1, 32), 1)
    ox = ((ci // (KERNEL_SIZE * KERNEL_SIZE)).astype(jnp.float32) - half) * DILATION
    oy = (((ci // KERNEL_SIZE) % KERNEL_SIZE).astype(jnp.float32) - half) * DILATION
    oz = ((ci % KERNEL_SIZE).astype(jnp.float32) - half) * DILATION

    dg = gd_ref[...]
    far = jnp.where(nb_ref[...] < 0, 1e3, 0.0).astype(jnp.float32)
    d0 = (q_ref[:, 0:1] + far) - gl_ref[:, 0:1]
    d1 = q_ref[:, 1:2] - gl_ref[:, 1:2]
    d2 = q_ref[:, 2:3] - gl_ref[:, 2:3]
    e0 = d0 + ox
    e1 = d1 + oy
    e2 = d2 + oz
    dist2 = e0 * e0 + e1 * e1 + e2 * e2
    dist = jnp.sqrt(dist2 + 1e-12)
    t = jnp.maximum(1.0 - dist * (1.0 / RADIUS), 0.0)
    w = norm * t * t * t

    nc32 = w.shape[1]
    wide = NCELLS * c
    widep = 128 * ((wide + 127) // 128)
    row_i = lax.broadcasted_iota(jnp.int32, (nc32, widep), 0)
    col_i = lax.broadcasted_iota(jnp.int32, (nc32, widep), 1)
    rep = jnp.where((col_i // c == row_i) & (col_i < wide), 1.0, 0.0)
    rep = rep.astype(jnp.float32)

    hi = w.astype(jnp.bfloat16).astype(jnp.float32)
    lo = w - hi
    wexp = (jnp.dot(hi, rep, preferred_element_type=jnp.float32)
            + jnp.dot(lo, rep, preferred_element_type=jnp.float32))

    dg4 = jnp.concatenate([dg, dg, dg, dg], axis=1)
    dgt = jnp.concatenate([dg4] * (widep // 128), axis=1)

    p = wexp * dgt
    interp = p.reshape(mb, k, widep).sum(axis=1)
    acc = jnp.dot(interp, w2_ref[...], preferred_element_type=jnp.float32,
                  precision=jax.lax.Precision.HIGHEST)
    o_ref[...] = acc + b_ref[...]


def kernel(locs, data, neighbors, qlocs, weight, bias):
    b, n, d = locs.shape
    _, m, k = neighbors.shape
    c = data.shape[2]
    o = weight.shape[0]
    lw = 16
    r = b * m * k
    bm = b * m

    data2 = data.reshape(b * n, c)
    ploc = jnp.concatenate(
        [locs, jnp.zeros((b, n, lw - d), jnp.float32)], axis=-1).reshape(b * n, lw)

    nb = neighbors.astype(jnp.int32)
    base = (jnp.arange(b, dtype=jnp.int32) * n)[:, None, None]
    flat_idx = (jnp.clip(nb, 0, n - 1) + base).reshape(r)
    nbf = nb.reshape(r, 1)

    q4 = jnp.concatenate([qlocs, jnp.zeros((b, m, 1), jnp.float32)], axis=-1)
    qrep = jnp.broadcast_to(q4[:, :, None, :], (b, m, k, 4)).reshape(r, 4)

    gd, gl = _make_sc_gather(r, c, lw)(data2, ploc, flat_idx)

    w2p = jnp.transpose(weight, (2, 1, 0)).reshape(NCELLS * c, o)
    widep = 128 * ((NCELLS * c + 127) // 128)
    w2p = jnp.concatenate(
        [w2p, jnp.zeros((widep - NCELLS * c, o), jnp.float32)], axis=0)
    mb = 128
    out2 = pl.pallas_call(
        functools.partial(_tc_body, k, c, mb),
        grid=(bm // mb,),
        in_specs=[
            pl.BlockSpec((mb * k, c), lambda i: (i, 0)),
            pl.BlockSpec((mb * k, lw), lambda i: (i, 0)),
            pl.BlockSpec((mb * k, 4), lambda i: (i, 0)),
            pl.BlockSpec((mb * k, 1), lambda i: (i, 0)),
            pl.BlockSpec((widep, o), lambda i: (0, 0)),
            pl.BlockSpec((1, o), lambda i: (0, 0)),
        ],
        out_specs=pl.BlockSpec((mb, o), lambda i: (i, 0)),
        out_shape=jax.ShapeDtypeStruct((bm, o), jnp.float32),
    )(gd, gl, qrep, nbf, w2p, bias.reshape(1, o))

    return out2.reshape(b, m, o)


# speedup vs baseline: 1.4130x; 1.3994x over previous
import functools

import jax
import jax.numpy as jnp
from jax import lax
from jax.experimental import pallas as pl
from jax.experimental.pallas import tpu as pltpu
from jax.experimental.pallas import tpu_sc as plsc

RADIUS = 0.1
KERNEL_SIZE = 3
DILATION = 0.05
NDIM = 3
NCELLS = KERNEL_SIZE ** NDIM


def _make_sc_gather(n_idx, cw, lw, kk):
    info = plsc.get_sparse_core_info()
    nc, ns = info.num_cores, info.num_subcores
    nw = nc * ns
    ch = 128
    chq = ch // kk
    assert n_idx % (nw * ch) == 0
    per_w = n_idx // nw
    iters = per_w // ch
    nbuf = 8
    assert iters % nbuf == 0
    mesh = plsc.VectorSubcoreMesh(core_axis_name="c", subcore_axis_name="s")

    @functools.partial(
        pl.kernel,
        mesh=mesh,
        out_type=(jax.ShapeDtypeStruct((n_idx, cw), jnp.float32),
                  jax.ShapeDtypeStruct((n_idx, 2 * lw), jnp.float32)),
        scratch_types=[
            pltpu.VMEM((per_w,), jnp.int32),
            pltpu.VMEM((nbuf, ch, cw), jnp.float32),
            pltpu.VMEM((nbuf, ch, lw), jnp.float32),
            pltpu.VMEM((nbuf, ch, 2 * lw), jnp.float32),
            pltpu.VMEM((nbuf, chq, lw), jnp.float32),
        ] + [pltpu.SemaphoreType.DMA] * (2 * nbuf),
        compiler_params=pltpu.CompilerParams(use_tc_tiling_on_sc=False),
    )
    def gather(data_hbm, ploc_hbm, q_hbm, idx_hbm, gd_hbm, gs_hbm,
               idx_v, rows_d, rows_l, rows_s, qbuf, *sems):
        gsem = sems[0:nbuf]
        ssem = sems[nbuf:2 * nbuf]
        wid = lax.axis_index("s") * nc + lax.axis_index("c")
        base = wid * per_w
        baseq = wid * (per_w // kk)
        pltpu.sync_copy(idx_hbm.at[pl.ds(base, per_w)], idx_v)

        def issue(i, s):
            sl = idx_v.at[pl.ds(i * ch, ch)]
            pltpu.async_copy(data_hbm.at[sl], rows_d.at[s], gsem[s])
            pltpu.async_copy(ploc_hbm.at[sl], rows_l.at[s], gsem[s])
            pltpu.async_copy(q_hbm.at[pl.ds(baseq + i * chq, chq)],
                             qbuf.at[s], gsem[s])

        for s in range(nbuf):
            issue(s, s)

        def blk(ib, carry):
            i0 = ib * nbuf
            for s in range(nbuf):
                i = i0 + s
                sl = idx_v.at[pl.ds(i * ch, ch)]
                pltpu.make_async_copy(data_hbm.at[sl], rows_d.at[s], gsem[s]).wait()
                pltpu.make_async_copy(ploc_hbm.at[sl], rows_l.at[s], gsem[s]).wait()
                pltpu.make_async_copy(q_hbm.at[pl.ds(baseq + i * chq, chq)],
                                      qbuf.at[s], gsem[s]).wait()
                rl = rows_l.at[s]
                rs = rows_s.at[s]
                qb = qbuf.at[s]
                for j in range(chq):
                    qv = qb[j, :]
                    for t in range(kk):
                        row = j * kk + t
                        rs[row, 0:lw] = qv - rl[row, :]
                st_d = pltpu.async_copy(
                    rows_d.at[s], gd_hbm.at[pl.ds(base + i * ch, ch)], ssem[s])
                st_l = pltpu.async_copy(
                    rows_s.at[s], gs_hbm.at[pl.ds(base + i * ch, ch)], ssem[s])
                st_d.wait()
                st_l.wait()

                @pl.when(ib < (iters // nbuf) - 1)
                def _():
                    issue(i + nbuf, s)
            return carry

        lax.fori_loop(0, iters // nbuf, blk, 0)

    return gather


def _tc_body(k, c, mb, gd4_ref, gs4_ref, w2_ref, b_ref, o_ref):
    norm = 1.0 / (RADIUS ** 3)
    half = (KERNEL_SIZE - 1) / 2.0
    ci = lax.broadcasted_iota(jnp.int32, (1, 32), 1)
    ox = ((ci // (KERNEL_SIZE * KERNEL_SIZE)).astype(jnp.float32) - half) * DILATION
    oy = (((ci // KERNEL_SIZE) % KERNEL_SIZE).astype(jnp.float32) - half) * DILATION
    oz = ((ci % KERNEL_SIZE).astype(jnp.float32) - half) * DILATION

    wide = NCELLS * c
    widep = 128 * ((wide + 127) // 128)
    row_i = lax.broadcasted_iota(jnp.int32, (32, widep), 0)
    col_i = lax.broadcasted_iota(jnp.int32, (32, widep), 1)
    rep = jnp.where((col_i // c == row_i) & (col_i < wide), 1.0, 0.0)
    rep = rep.astype(jnp.float32)

    g = gd4_ref[...]
    sarr = gs4_ref[...]
    psum = None
    for kk in range(4):
        s0 = sarr[:, 32 * kk:32 * kk + 1]
        s1 = sarr[:, 32 * kk + 1:32 * kk + 2]
        s2 = sarr[:, 32 * kk + 2:32 * kk + 3]
        e0 = s0 + ox
        e1 = s1 + oy
        e2 = s2 + oz
        dist2 = e0 * e0 + e1 * e1 + e2 * e2
        dist = jnp.sqrt(dist2 + 1e-12)
        t = jnp.maximum(1.0 - dist * (1.0 / RADIUS), 0.0)
        w = norm * t * t * t

        hi = w.astype(jnp.bfloat16).astype(jnp.float32)
        lo = w - hi
        wexp = (jnp.dot(hi, rep, preferred_element_type=jnp.float32)
                + jnp.dot(lo, rep, preferred_element_type=jnp.float32))

        dgkk = g[:, 32 * kk:32 * kk + c]
        dg4 = jnp.concatenate([dgkk] * 4, axis=1)
        dgt = jnp.concatenate([dg4] * (widep // 128), axis=1)
        pk = wexp * dgt
        psum = pk if psum is None else psum + pk

    interp = psum.reshape(mb, k // 4, widep).sum(axis=1)
    acc = jnp.dot(interp, w2_ref[...], preferred_element_type=jnp.float32,
                  precision=jax.lax.Precision.HIGHEST)
    o_ref[...] = acc + b_ref[...]


def kernel(locs, data, neighbors, qlocs, weight, bias):
    b, n, d = locs.shape
    _, m, k = neighbors.shape
    c = data.shape[2]
    o = weight.shape[0]
    lw = 16
    r = b * m * k
    bm = b * m
    bn = b * n

    data2 = jnp.concatenate(
        [data.reshape(bn, c), jnp.zeros((8, c), jnp.float32)], axis=0)
    ploc = jnp.concatenate(
        [locs, jnp.zeros((b, n, lw - d), jnp.float32)], axis=-1).reshape(bn, lw)
    ploc = jnp.concatenate(
        [ploc, jnp.full((8, lw), 2000.0, jnp.float32)], axis=0)

    nb = neighbors.astype(jnp.int32)
    base = (jnp.arange(b, dtype=jnp.int32) * n)[:, None, None]
    flat_idx = jnp.where(nb < 0, bn, nb + base).reshape(r)

    q16 = jnp.concatenate(
        [qlocs, jnp.zeros((b, m, lw - d), jnp.float32)], axis=-1).reshape(bm, lw)

    gd, gs = _make_sc_gather(r, c, lw, k)(data2, ploc, q16, flat_idx)
    gd4 = gd.reshape(r // 4, 128)
    gs4 = gs.reshape(r // 4, 128)

    w2p = jnp.transpose(weight, (2, 1, 0)).reshape(NCELLS * c, o)
    widep = 128 * ((NCELLS * c + 127) // 128)
    w2p = jnp.concatenate(
        [w2p, jnp.zeros((widep - NCELLS * c, o), jnp.float32)], axis=0)
    mb = 128
    out2 = pl.pallas_call(
        functools.partial(_tc_body, k, c, mb),
        grid=(bm // mb,),
        in_specs=[
            pl.BlockSpec((mb * k // 4, 128), lambda i: (i, 0)),
            pl.BlockSpec((mb * k // 4, 128), lambda i: (i, 0)),
            pl.BlockSpec((widep, o), lambda i: (0, 0)),
            pl.BlockSpec((1, o), lambda i: (0, 0)),
        ],
        out_specs=pl.BlockSpec((mb, o), lambda i: (i, 0)),
        out_shape=jax.ShapeDtypeStruct((bm, o), jnp.float32),
    )(gd4, gs4, w2p, bias.reshape(1, o))

    return out2.reshape(b, m, o)


# speedup vs baseline: 1.4456x; 1.0231x over previous
import functools

import jax
import jax.numpy as jnp
from jax import lax
from jax.experimental import pallas as pl
from jax.experimental.pallas import tpu as pltpu
from jax.experimental.pallas import tpu_sc as plsc

RADIUS = 0.1
KERNEL_SIZE = 3
DILATION = 0.05
NDIM = 3
NCELLS = KERNEL_SIZE ** NDIM


def _make_sc_gather(n_idx, cw, lw, kk):
    info = plsc.get_sparse_core_info()
    nc, ns = info.num_cores, info.num_subcores
    nw = nc * ns
    ch = 128
    chq = ch // kk
    assert n_idx % (nw * ch) == 0
    per_w = n_idx // nw
    iters = per_w // ch
    nbuf = 8
    assert iters % nbuf == 0
    mesh = plsc.VectorSubcoreMesh(core_axis_name="c", subcore_axis_name="s")

    @functools.partial(
        pl.kernel,
        mesh=mesh,
        out_type=(jax.ShapeDtypeStruct((n_idx, cw), jnp.float32),
                  jax.ShapeDtypeStruct((n_idx, 2 * lw), jnp.float32)),
        scratch_types=[
            pltpu.VMEM((per_w,), jnp.int32),
            pltpu.VMEM((nbuf, ch, cw), jnp.float32),
            pltpu.VMEM((nbuf, ch, lw), jnp.float32),
            pltpu.VMEM((nbuf, ch, 2 * lw), jnp.float32),
            pltpu.VMEM((nbuf, chq, lw), jnp.float32),
        ] + [pltpu.SemaphoreType.DMA] * (2 * nbuf),
        compiler_params=pltpu.CompilerParams(use_tc_tiling_on_sc=False),
    )
    def gather(data_hbm, ploc_hbm, q_hbm, idx_hbm, gd_hbm, gs_hbm,
               idx_v, rows_d, rows_l, rows_s, qbuf, *sems):
        gsem = sems[0:nbuf]
        ssem = sems[nbuf:2 * nbuf]
        wid = lax.axis_index("s") * nc + lax.axis_index("c")
        base = wid * per_w
        baseq = wid * (per_w // kk)
        pltpu.sync_copy(idx_hbm.at[pl.ds(base, per_w)], idx_v)

        def issue(i, s):
            sl = idx_v.at[pl.ds(i * ch, ch)]
            pltpu.async_copy(data_hbm.at[sl], rows_d.at[s], gsem[s])
            pltpu.async_copy(ploc_hbm.at[sl], rows_l.at[s], gsem[s])
            pltpu.async_copy(q_hbm.at[pl.ds(baseq + i * chq, chq)],
                             qbuf.at[s], gsem[s])

        for s in range(nbuf):
            issue(s, s)

        def blk(ib, carry):
            i0 = ib * nbuf
            for s in range(nbuf):
                i = i0 + s
                sl = idx_v.at[pl.ds(i * ch, ch)]
                pltpu.make_async_copy(data_hbm.at[sl], rows_d.at[s], gsem[s]).wait()
                pltpu.make_async_copy(ploc_hbm.at[sl], rows_l.at[s], gsem[s]).wait()
                pltpu.make_async_copy(q_hbm.at[pl.ds(baseq + i * chq, chq)],
                                      qbuf.at[s], gsem[s]).wait()
                rl = rows_l.at[s]
                rs = rows_s.at[s]
                qb = qbuf.at[s]
                for j in range(chq):
                    qv = qb[j, :]
                    for t in range(kk):
                        row = j * kk + t
                        rs[row, 0:lw] = qv - rl[row, :]
                st_d = pltpu.async_copy(
                    rows_d.at[s], gd_hbm.at[pl.ds(base + i * ch, ch)], ssem[s])
                st_l = pltpu.async_copy(
                    rows_s.at[s], gs_hbm.at[pl.ds(base + i * ch, ch)], ssem[s])
                st_d.wait()
                st_l.wait()

                @pl.when(ib < (iters // nbuf) - 1)
                def _():
                    issue(i + nbuf, s)
            return carry

        lax.fori_loop(0, iters // nbuf, blk, 0)

    return gather


def _tc_body(k, c, mb, gd4_ref, gs4_ref, rep_ref, w2_ref, b_ref, o_ref):
    norm = 1.0 / (RADIUS ** 3)
    half = (KERNEL_SIZE - 1) / 2.0
    ci = lax.broadcasted_iota(jnp.int32, (1, 32), 1)
    ox = ((ci // (KERNEL_SIZE * KERNEL_SIZE)).astype(jnp.float32) - half) * DILATION
    oy = (((ci // KERNEL_SIZE) % KERNEL_SIZE).astype(jnp.float32) - half) * DILATION
    oz = ((ci % KERNEL_SIZE).astype(jnp.float32) - half) * DILATION

    wide = NCELLS * c
    widep = 128 * ((wide + 127) // 128)
    rep = rep_ref[...]

    g = gd4_ref[...]
    sarr = gs4_ref[...]
    psum = None
    for kk in range(4):
        s0 = sarr[:, 32 * kk:32 * kk + 1]
        s1 = sarr[:, 32 * kk + 1:32 * kk + 2]
        s2 = sarr[:, 32 * kk + 2:32 * kk + 3]
        e0 = s0 + ox
        e1 = s1 + oy
        e2 = s2 + oz
        dist2 = e0 * e0 + e1 * e1 + e2 * e2
        dist = jnp.sqrt(dist2 + 1e-12)
        t = jnp.maximum(1.0 - dist * (1.0 / RADIUS), 0.0)
        w = norm * t * t * t

        hi = w.astype(jnp.bfloat16).astype(jnp.float32)
        lo = w - hi
        wexp = (jnp.dot(hi, rep, preferred_element_type=jnp.float32)
                + jnp.dot(lo, rep, preferred_element_type=jnp.float32))

        dgkk = g[:, 32 * kk:32 * kk + c]
        dg4 = jnp.concatenate([dgkk] * 4, axis=1)
        dgt = jnp.concatenate([dg4] * (widep // 128), axis=1)
        pk = wexp * dgt
        psum = pk if psum is None else psum + pk

    interp = psum.reshape(mb, k // 4, widep).sum(axis=1)
    acc = jnp.dot(interp, w2_ref[...], preferred_element_type=jnp.float32,
                  precision=jax.lax.Precision.HIGHEST)
    o_ref[...] = acc + b_ref[...]


def kernel(locs, data, neighbors, qlocs, weight, bias):
    b, n, d = locs.shape
    _, m, k = neighbors.shape
    c = data.shape[2]
    o = weight.shape[0]
    lw = 16
    r = b * m * k
    bm = b * m
    bn = b * n

    data2 = jnp.concatenate(
        [data.reshape(bn, c), jnp.zeros((8, c), jnp.float32)], axis=0)
    ploc = jnp.concatenate(
        [locs, jnp.zeros((b, n, lw - d), jnp.float32)], axis=-1).reshape(bn, lw)
    ploc = jnp.concatenate(
        [ploc, jnp.full((8, lw), 2000.0, jnp.float32)], axis=0)

    nb = neighbors.astype(jnp.int32)
    base = (jnp.arange(b, dtype=jnp.int32) * n)[:, None, None]
    flat_idx = jnp.where(nb < 0, bn, nb + base).reshape(r)

    q16 = jnp.concatenate(
        [qlocs, jnp.zeros((b, m, lw - d), jnp.float32)], axis=-1).reshape(bm, lw)

    gd, gs = _make_sc_gather(r, c, lw, k)(data2, ploc, q16, flat_idx)
    gd4 = gd.reshape(r // 4, 128)
    gs4 = gs.reshape(r // 4, 128)

    row_i = jnp.arange(32, dtype=jnp.int32)[:, None]
    widep0 = 128 * ((NCELLS * c + 127) // 128)
    col_i = jnp.arange(widep0, dtype=jnp.int32)[None, :]
    rep = ((col_i // c == row_i) & (col_i < NCELLS * c)).astype(jnp.float32)

    w2p = jnp.transpose(weight, (2, 1, 0)).reshape(NCELLS * c, o)
    widep = 128 * ((NCELLS * c + 127) // 128)
    w2p = jnp.concatenate(
        [w2p, jnp.zeros((widep - NCELLS * c, o), jnp.float32)], axis=0)
    mb = 256
    out2 = pl.pallas_call(
        functools.partial(_tc_body, k, c, mb),
        grid=(bm // mb,),
        in_specs=[
            pl.BlockSpec((mb * k // 4, 128), lambda i: (i, 0)),
            pl.BlockSpec((mb * k // 4, 128), lambda i: (i, 0)),
            pl.BlockSpec((32, widep), lambda i: (0, 0)),
            pl.BlockSpec((widep, o), lambda i: (0, 0)),
            pl.BlockSpec((1, o), lambda i: (0, 0)),
        ],
        out_specs=pl.BlockSpec((mb, o), lambda i: (i, 0)),
        out_shape=jax.ShapeDtypeStruct((bm, o), jnp.float32),
    )(gd4, gs4, rep, w2p, bias.reshape(1, o))

    return out2.reshape(b, m, o)


# speedup vs baseline: 1.7924x; 1.2399x over previous
import functools

import numpy as _np
import jax
import jax.numpy as jnp
from jax import lax
from jax.experimental import pallas as pl
from jax.experimental.pallas import tpu as pltpu
from jax.experimental.pallas import tpu_sc as plsc

RADIUS = 0.1
KERNEL_SIZE = 3
DILATION = 0.05
NDIM = 3
NCELLS = KERNEL_SIZE ** NDIM


def _make_sc_gather(n_idx, cw, lw, kk):
    info = plsc.get_sparse_core_info()
    nc, ns = info.num_cores, info.num_subcores
    nw = nc * ns
    ch = 128
    chq = ch // kk
    assert n_idx % (nw * ch) == 0
    per_w = n_idx // nw
    iters = per_w // ch
    nbuf = 8
    assert iters % nbuf == 0
    mesh = plsc.VectorSubcoreMesh(core_axis_name="c", subcore_axis_name="s")

    @functools.partial(
        pl.kernel,
        mesh=mesh,
        out_type=(jax.ShapeDtypeStruct((n_idx, cw), jnp.float32),
                  jax.ShapeDtypeStruct((n_idx, 2 * lw), jnp.float32)),
        scratch_types=[
            pltpu.VMEM((per_w,), jnp.int32),
            pltpu.VMEM((nbuf, ch, cw), jnp.float32),
            pltpu.VMEM((nbuf, ch, lw), jnp.float32),
            pltpu.VMEM((nbuf, ch, 2 * lw), jnp.float32),
            pltpu.VMEM((nbuf, chq, lw), jnp.float32),
        ] + [pltpu.SemaphoreType.DMA] * (2 * nbuf),
        compiler_params=pltpu.CompilerParams(use_tc_tiling_on_sc=False),
    )
    def gather(data_hbm, ploc_hbm, q_hbm, idx_hbm, gd_hbm, gs_hbm,
               idx_v, rows_d, rows_l, rows_s, qbuf, *sems):
        gsem = sems[0:nbuf]
        ssem = sems[nbuf:2 * nbuf]
        wid = lax.axis_index("s") * nc + lax.axis_index("c")
        base = wid * per_w
        baseq = wid * (per_w // kk)
        pltpu.sync_copy(idx_hbm.at[pl.ds(base, per_w)], idx_v)

        zv = jnp.zeros((lw,), jnp.float32)
        for s0_ in range(nbuf):
            for row_ in range(ch):
                rows_s.at[s0_][row_, lw:2 * lw] = zv

        def issue(i, s):
            sl = idx_v.at[pl.ds(i * ch, ch)]
            pltpu.async_copy(data_hbm.at[sl], rows_d.at[s], gsem[s])
            pltpu.async_copy(ploc_hbm.at[sl], rows_l.at[s], gsem[s])
            pltpu.async_copy(q_hbm.at[pl.ds(baseq + i * chq, chq)],
                             qbuf.at[s], gsem[s])

        for s in range(nbuf):
            issue(s, s)

        def blk(ib, carry):
            i0 = ib * nbuf
            for s in range(nbuf):
                i = i0 + s
                sl = idx_v.at[pl.ds(i * ch, ch)]
                pltpu.make_async_copy(data_hbm.at[sl], rows_d.at[s], gsem[s]).wait()
                pltpu.make_async_copy(ploc_hbm.at[sl], rows_l.at[s], gsem[s]).wait()
                pltpu.make_async_copy(q_hbm.at[pl.ds(baseq + i * chq, chq)],
                                      qbuf.at[s], gsem[s]).wait()
                rl = rows_l.at[s]
                rs = rows_s.at[s]
                qb = qbuf.at[s]
                for j in range(chq):
                    qv = qb[j, :]
                    for t in range(kk):
                        row = j * kk + t
                        rs[row, 0:lw] = qv - rl[row, :]
                st_d = pltpu.async_copy(
                    rows_d.at[s], gd_hbm.at[pl.ds(base + i * ch, ch)], ssem[s])
                st_l = pltpu.async_copy(
                    rows_s.at[s], gs_hbm.at[pl.ds(base + i * ch, ch)], ssem[s])
                st_d.wait()
                st_l.wait()

                @pl.when(ib < (iters // nbuf) - 1)
                def _():
                    issue(i + nbuf, s)
            return carry

        lax.fori_loop(0, iters // nbuf, blk, 0)

    return gather


def _tc_body(k, c, mb, gd4_ref, gs4_ref, mat_ref, offn_ref, rep_ref,
             w2_ref, b_ref, o_ref):
    norm = 1.0 / (RADIUS ** 3)
    wide = NCELLS * c
    widep = 128 * ((wide + 127) // 128)
    rep = rep_ref[...]

    g = gd4_ref[...]
    sarr = gs4_ref[...]

    u = sarr * 0.1
    x = jnp.concatenate([u, sarr * sarr], axis=1)
    xhi = x.astype(jnp.bfloat16).astype(jnp.float32)
    xlo = x - xhi
    mat = mat_ref[...]
    dist2 = (jnp.dot(xhi, mat, preferred_element_type=jnp.float32)
             + jnp.dot(xlo, mat, preferred_element_type=jnp.float32)
             + offn_ref[...])
    dist2 = jnp.maximum(dist2, 0.0)
    dist = jnp.sqrt(dist2 + 1e-12)
    t = jnp.maximum(1.0 - dist * (1.0 / RADIUS), 0.0)
    w_all = norm * t * t * t

    whi = w_all.astype(jnp.bfloat16).astype(jnp.float32)
    wlo = w_all - whi
    psum = None
    for kk in range(4):
        wexp = (jnp.dot(whi[:, 32 * kk:32 * kk + 32], rep,
                        preferred_element_type=jnp.float32)
                + jnp.dot(wlo[:, 32 * kk:32 * kk + 32], rep,
                          preferred_element_type=jnp.float32))
        dgkk = g[:, 32 * kk:32 * kk + c]
        dg4 = jnp.concatenate([dgkk] * 4, axis=1)
        dgt = jnp.concatenate([dg4] * (widep // 128), axis=1)
        pk = wexp * dgt
        psum = pk if psum is None else psum + pk

    phi = psum.astype(jnp.bfloat16).astype(jnp.float32)
    plo = psum - phi
    w2 = w2_ref[...]
    outr = (jnp.dot(phi, w2, preferred_element_type=jnp.float32)
            + jnp.dot(plo, w2, preferred_element_type=jnp.float32))
    osum = outr.reshape(mb, k // 4, w2.shape[1]).sum(axis=1)
    o_ref[...] = osum + b_ref[...]


def kernel(locs, data, neighbors, qlocs, weight, bias):
    b, n, d = locs.shape
    _, m, k = neighbors.shape
    c = data.shape[2]
    o = weight.shape[0]
    lw = 16
    r = b * m * k
    bm = b * m
    bn = b * n

    data2 = jnp.concatenate(
        [data.reshape(bn, c), jnp.zeros((8, c), jnp.float32)], axis=0)
    ploc = jnp.concatenate(
        [locs, jnp.zeros((b, n, lw - d), jnp.float32)], axis=-1).reshape(bn, lw)
    ploc = jnp.concatenate(
        [ploc, jnp.full((8, lw), 2000.0, jnp.float32)], axis=0)

    nb = neighbors.astype(jnp.int32)
    base = (jnp.arange(b, dtype=jnp.int32) * n)[:, None, None]
    flat_idx = jnp.where(nb < 0, bn, nb + base).reshape(r)

    q16 = jnp.concatenate(
        [qlocs, jnp.zeros((b, m, lw - d), jnp.float32)], axis=-1).reshape(bm, lw)

    gd, gs = _make_sc_gather(r, c, lw, k)(data2, ploc, q16, flat_idx)
    gd4 = gd.reshape(r // 4, 128)
    gs4 = gs.reshape(r // 4, 128)

    row_i = jnp.arange(32, dtype=jnp.int32)[:, None]
    widep0 = 128 * ((NCELLS * c + 127) // 128)
    col_i = jnp.arange(widep0, dtype=jnp.int32)[None, :]
    rep = ((col_i // c == row_i) & (col_i < NCELLS * c)).astype(jnp.float32)

    cell = _np.arange(32)
    digs = _np.stack([cell // 9, (cell // 3) % 3, cell % 3]) - 1
    mat_np = _np.zeros((256, 128), _np.float32)
    offn_np = _np.zeros((1, 128), _np.float32)
    for kk4 in range(4):
        for dd in range(3):
            mat_np[kk4 * 32 + dd, kk4 * 32 + cell] = digs[dd]
            mat_np[128 + kk4 * 32 + dd, kk4 * 32 + cell] = 1.0
        offn_np[0, kk4 * 32 + cell] = ((digs * DILATION) ** 2).sum(axis=0)
    mat = jnp.asarray(mat_np)
    offn = jnp.asarray(offn_np)

    w2p = jnp.transpose(weight, (2, 1, 0)).reshape(NCELLS * c, o)
    widep = 128 * ((NCELLS * c + 127) // 128)
    w2p = jnp.concatenate(
        [w2p, jnp.zeros((widep - NCELLS * c, o), jnp.float32)], axis=0)
    mb = 256
    out2 = pl.pallas_call(
        functools.partial(_tc_body, k, c, mb),
        grid=(bm // mb,),
        in_specs=[
            pl.BlockSpec((mb * k // 4, 128), lambda i: (i, 0)),
            pl.BlockSpec((mb * k // 4, 128), lambda i: (i, 0)),
            pl.BlockSpec((256, 128), lambda i: (0, 0)),
            pl.BlockSpec((1, 128), lambda i: (0, 0)),
            pl.BlockSpec((32, widep), lambda i: (0, 0)),
            pl.BlockSpec((widep, o), lambda i: (0, 0)),
            pl.BlockSpec((1, o), lambda i: (0, 0)),
        ],
        out_specs=pl.BlockSpec((mb, o), lambda i: (i, 0)),
        out_shape=jax.ShapeDtypeStruct((bm, o), jnp.float32),
    )(gd4, gs4, mat, offn, rep, w2p, bias.reshape(1, o))

    return out2.reshape(b, m, o)


# speedup vs baseline: 2.3508x; 1.3115x over previous
import functools

import numpy as _np
import jax
import jax.numpy as jnp
from jax import lax
from jax.experimental import pallas as pl
from jax.experimental.pallas import tpu as pltpu
from jax.experimental.pallas import tpu_sc as plsc

RADIUS = 0.1
KERNEL_SIZE = 3
DILATION = 0.05
NDIM = 3
NCELLS = KERNEL_SIZE ** NDIM


def _make_sc_gather(n_idx, cw, lw, kk):
    info = plsc.get_sparse_core_info()
    nc, ns = info.num_cores, info.num_subcores
    nw = nc * ns
    ch = 128
    chq = ch // kk
    assert n_idx % (nw * ch) == 0
    per_w = n_idx // nw
    iters = per_w // ch
    nbuf = 8
    assert iters % nbuf == 0
    mesh = plsc.VectorSubcoreMesh(core_axis_name="c", subcore_axis_name="s")

    @functools.partial(
        pl.kernel,
        mesh=mesh,
        out_type=(jax.ShapeDtypeStruct((n_idx, cw), jnp.float32),
                  jax.ShapeDtypeStruct((n_idx, 2 * lw), jnp.float32)),
        scratch_types=[
            pltpu.VMEM((per_w,), jnp.int32),
            pltpu.VMEM((nbuf, ch, cw), jnp.float32),
            pltpu.VMEM((nbuf, ch, lw), jnp.float32),
            pltpu.VMEM((nbuf, ch, 2 * lw), jnp.float32),
            pltpu.VMEM((nbuf, chq, lw), jnp.float32),
        ] + [pltpu.SemaphoreType.DMA] * (2 * nbuf),
        compiler_params=pltpu.CompilerParams(use_tc_tiling_on_sc=False),
    )
    def gather(data_hbm, ploc_hbm, q_hbm, idx_hbm, gd_hbm, gs_hbm,
               idx_v, rows_d, rows_l, rows_s, qbuf, *sems):
        gsem = sems[0:nbuf]
        ssem = sems[nbuf:2 * nbuf]
        wid = lax.axis_index("s") * nc + lax.axis_index("c")
        base = wid * per_w
        baseq = wid * (per_w // kk)
        pltpu.sync_copy(idx_hbm.at[pl.ds(base, per_w)], idx_v)

        zv = jnp.zeros((lw,), jnp.float32)
        for s0_ in range(nbuf):
            for row_ in range(ch):
                rows_s.at[s0_][row_, lw:2 * lw] = zv

        def issue(i, s):
            sl = idx_v.at[pl.ds(i * ch, ch)]
            pltpu.async_copy(data_hbm.at[sl], rows_d.at[s], gsem[s])
            pltpu.async_copy(ploc_hbm.at[sl], rows_l.at[s], gsem[s])
            pltpu.async_copy(q_hbm.at[pl.ds(baseq + i * chq, chq)],
                             qbuf.at[s], gsem[s])

        for s in range(nbuf):
            issue(s, s)

        def blk(ib, carry):
            i0 = ib * nbuf
            for s in range(nbuf):
                i = i0 + s
                sl = idx_v.at[pl.ds(i * ch, ch)]
                pltpu.make_async_copy(data_hbm.at[sl], rows_d.at[s], gsem[s]).wait()
                pltpu.make_async_copy(ploc_hbm.at[sl], rows_l.at[s], gsem[s]).wait()
                pltpu.make_async_copy(q_hbm.at[pl.ds(baseq + i * chq, chq)],
                                      qbuf.at[s], gsem[s]).wait()
                rl = rows_l.at[s]
                rs = rows_s.at[s]
                qb = qbuf.at[s]
                for j in range(chq):
                    qv = qb[j, :]
                    for t in range(kk):
                        row = j * kk + t
                        rs[row, 0:lw] = qv - rl[row, :]
                st_d = pltpu.async_copy(
                    rows_d.at[s], gd_hbm.at[pl.ds(base + i * ch, ch)], ssem[s])
                st_l = pltpu.async_copy(
                    rows_s.at[s], gs_hbm.at[pl.ds(base + i * ch, ch)], ssem[s])
                st_d.wait()
                st_l.wait()

                @pl.when(ib < (iters // nbuf) - 1)
                def _():
                    issue(i + nbuf, s)
            return carry

        lax.fori_loop(0, iters // nbuf, blk, 0)

    return gather


def _tc_body(k, c, mb, gd4_ref, gs4_ref, mat_ref, offn_ref, rep_ref,
             w2_ref, b_ref, o_ref):
    norm = 1.0 / (RADIUS ** 3)
    wide = NCELLS * c
    widep = 128 * ((wide + 127) // 128)
    rep = rep_ref[...]

    g = gd4_ref[...]
    sarr = gs4_ref[...]

    u = sarr * 0.1
    x = jnp.concatenate([u, sarr * sarr], axis=1)
    xhi = x.astype(jnp.bfloat16).astype(jnp.float32)
    xlo = x - xhi
    mat = mat_ref[...]
    dist2 = (jnp.dot(xhi, mat, preferred_element_type=jnp.float32)
             + jnp.dot(xlo, mat, preferred_element_type=jnp.float32)
             + offn_ref[...])
    dist2 = jnp.maximum(dist2, 0.0)
    dist = jnp.sqrt(dist2 + 1e-12)
    t = jnp.maximum(1.0 - dist * (1.0 / RADIUS), 0.0)
    w_all = norm * t * t * t

    psum = None
    for kk in range(4):
        wexp = jnp.dot(w_all[:, 32 * kk:32 * kk + 32], rep,
                       preferred_element_type=jnp.float32)
        dgkk = g[:, 32 * kk:32 * kk + c]
        dg4 = jnp.concatenate([dgkk] * 4, axis=1)
        dgt = jnp.concatenate([dg4] * (widep // 128), axis=1)
        pk = wexp * dgt
        psum = pk if psum is None else psum + pk

    phi = psum.astype(jnp.bfloat16).astype(jnp.float32)
    plo = psum - phi
    w2 = w2_ref[...]
    outr = (jnp.dot(phi, w2, preferred_element_type=jnp.float32)
            + jnp.dot(plo, w2, preferred_element_type=jnp.float32))
    osum = outr.reshape(mb, k // 4, w2.shape[1]).sum(axis=1)
    o_ref[...] = osum + b_ref[...]


def kernel(locs, data, neighbors, qlocs, weight, bias):
    b, n, d = locs.shape
    _, m, k = neighbors.shape
    c = data.shape[2]
    o = weight.shape[0]
    lw = 16
    r = b * m * k
    bm = b * m
    bn = b * n

    data2 = jnp.concatenate(
        [data.reshape(bn, c), jnp.zeros((8, c), jnp.float32)], axis=0)
    ploc = jnp.concatenate(
        [locs, jnp.zeros((b, n, lw - d), jnp.float32)], axis=-1).reshape(bn, lw)
    ploc = jnp.concatenate(
        [ploc, jnp.full((8, lw), 2000.0, jnp.float32)], axis=0)

    nb = neighbors.astype(jnp.int32)
    base = (jnp.arange(b, dtype=jnp.int32) * n)[:, None, None]
    flat_idx = jnp.where(nb < 0, bn, nb + base).reshape(r)

    q16 = jnp.concatenate(
        [qlocs, jnp.zeros((b, m, lw - d), jnp.float32)], axis=-1).reshape(bm, lw)

    gd, gs = _make_sc_gather(r, c, lw, k)(data2, ploc, q16, flat_idx)
    gd4 = gd.reshape(r // 4, 128)
    gs4 = gs.reshape(r // 4, 128)

    row_i = jnp.arange(32, dtype=jnp.int32)[:, None]
    widep0 = 128 * ((NCELLS * c + 127) // 128)
    col_i = jnp.arange(widep0, dtype=jnp.int32)[None, :]
    rep = ((col_i // c == row_i) & (col_i < NCELLS * c)).astype(jnp.float32)

    cell = _np.arange(32)
    digs = _np.stack([cell // 9, (cell // 3) % 3, cell % 3]) - 1
    mat_np = _np.zeros((256, 128), _np.float32)
    offn_np = _np.zeros((1, 128), _np.float32)
    for kk4 in range(4):
        for dd in range(3):
            mat_np[kk4 * 32 + dd, kk4 * 32 + cell] = digs[dd]
            mat_np[128 + kk4 * 32 + dd, kk4 * 32 + cell] = 1.0
        offn_np[0, kk4 * 32 + cell] = ((digs * DILATION) ** 2).sum(axis=0)
    mat = jnp.asarray(mat_np)
    offn = jnp.asarray(offn_np)

    w2p = jnp.transpose(weight, (2, 1, 0)).reshape(NCELLS * c, o)
    widep = 128 * ((NCELLS * c + 127) // 128)
    w2p = jnp.concatenate(
        [w2p, jnp.zeros((widep - NCELLS * c, o), jnp.float32)], axis=0)
    mb = 256
    out2 = pl.pallas_call(
        functools.partial(_tc_body, k, c, mb),
        grid=(bm // mb,),
        in_specs=[
            pl.BlockSpec((mb * k // 4, 128), lambda i: (i, 0)),
            pl.BlockSpec((mb * k // 4, 128), lambda i: (i, 0)),
            pl.BlockSpec((256, 128), lambda i: (0, 0)),
            pl.BlockSpec((1, 128), lambda i: (0, 0)),
            pl.BlockSpec((32, widep), lambda i: (0, 0)),
            pl.BlockSpec((widep, o), lambda i: (0, 0)),
            pl.BlockSpec((1, o), lambda i: (0, 0)),
        ],
        out_specs=pl.BlockSpec((mb, o), lambda i: (i, 0)),
        out_shape=jax.ShapeDtypeStruct((bm, o), jnp.float32),
    )(gd4, gs4, mat, offn, rep, w2p, bias.reshape(1, o))

    return out2.reshape(b, m, o)
